# baseline (speedup 1.0000x reference)
import functools

import jax
import jax.numpy as jnp
from jax import lax
from jax.experimental import pallas as pl
from jax.experimental.pallas import tpu as pltpu
from jax.experimental.pallas import tpu_sc as plsc

N = 32768
D = 64
K = 1024
TN = 4096
STEPS = N // TN
DP = 128

NW = 32
BPW = N // NW
CHUNK = 128
ROWS = BPW // CHUNK
GROUP = 256
NGROUP = BPW // GROUP


def _distance_body(x_ref, e_ref, idx_ref, tab_ref, loss_ref, perp_ref,
                   cnt_ref, acc_ref):
    step = pl.program_id(0)
    x = x_ref[...]
    e = e_ref[...]
    s = jnp.dot(-2.0 * x, e, preferred_element_type=jnp.float32)
    e2 = jnp.sum(e * e, axis=0, keepdims=True)
    half = s + e2
    m = jnp.min(half, axis=1, keepdims=True)
    ohm = half == m
    iota = lax.broadcasted_iota(jnp.int32, (TN, K), 1).astype(jnp.float32)
    idx = jnp.min(jnp.where(ohm, iota, float(K)), axis=1).astype(jnp.int32)
    oh = jnp.where(ohm, 1.0, 0.0)
    cnt_step = jnp.dot(jnp.full((1, TN), 1.0, jnp.float32), oh,
                       preferred_element_type=jnp.float32)
    part = jnp.sum(x * x) + jnp.sum(m)

    @pl.when(step == 0)
    def _init():
        cnt_ref[...] = jnp.zeros_like(cnt_ref)
        acc_ref[0] = 0.0
        et = jnp.swapaxes(e, 0, 1)
        tab_ref[...] = jnp.concatenate(
            [et, jnp.zeros((K, DP - D), jnp.float32)], axis=1)

    cnt_ref[...] += cnt_step
    acc_ref[0] += part
    idx_ref[...] = idx.reshape(TN // CHUNK, CHUNK)

    @pl.when(step == STEPS - 1)
    def _finish():
        loss_ref[...] = jnp.full((1, 1), acc_ref[0] / (N * D), jnp.float32)
        p = cnt_ref[...] * (1.0 / N)
        ent = jnp.sum(p * jnp.log(p + 1e-10))
        perp_ref[...] = jnp.full((1, 1), jnp.exp(-ent), jnp.float32)


def _distance_call(x, e):
    return pl.pallas_call(
        _distance_body,
        grid=(STEPS,),
        in_specs=[
            pl.BlockSpec((TN, D), lambda i: (i, 0)),
            pl.BlockSpec((D, K), lambda i: (0, 0)),
        ],
        out_specs=[
            pl.BlockSpec((TN // CHUNK, CHUNK), lambda i: (i, 0)),
            pl.BlockSpec((K, DP), lambda i: (0, 0)),
            pl.BlockSpec((1, 1), lambda i: (0, 0)),
            pl.BlockSpec((1, 1), lambda i: (0, 0)),
        ],
        out_shape=[
            jax.ShapeDtypeStruct((N // CHUNK, CHUNK), jnp.int32),
            jax.ShapeDtypeStruct((K, DP), jnp.float32),
            jax.ShapeDtypeStruct((1, 1), jnp.float32),
            jax.ShapeDtypeStruct((1, 1), jnp.float32),
        ],
        scratch_shapes=[
            pltpu.VMEM((1, K), jnp.float32),
            pltpu.SMEM((1,), jnp.float32),
        ],
    )(x, e)


def _make_sc_gather():
    mesh = plsc.VectorSubcoreMesh(core_axis_name="c", subcore_axis_name="s")

    @functools.partial(
        pl.kernel,
        mesh=mesh,
        compiler_params=pltpu.CompilerParams(use_tc_tiling_on_sc=False),
        out_type=jax.ShapeDtypeStruct((N, D), jnp.float32),
        scratch_types=[
            pltpu.VMEM((ROWS, CHUNK), jnp.int32),
            pltpu.VMEM((GROUP, DP), jnp.float32),
            pltpu.VMEM((GROUP, DP), jnp.float32),
            pltpu.VMEM((GROUP, DP), jnp.float32),
            pltpu.SemaphoreType.DMA,
            pltpu.SemaphoreType.DMA,
        ],
    )
    def gather_kernel(table_hbm, idx_hbm, out_hbm, idx_v, buf0, buf1, buf2,
                      gsem, osem):
        wid = lax.axis_index("s") * 2 + lax.axis_index("c")
        base = wid * BPW
        pltpu.sync_copy(idx_hbm.at[pl.ds(wid * ROWS, ROWS)], idx_v)
        bufs = [buf0, buf1, buf2]
        per_group = GROUP // CHUNK
        out_copies = [None] * NGROUP
        for g in range(NGROUP):
            buf = bufs[g % 3]
            if g >= 3:
                out_copies[g - 3].wait()
            gathers = [
                pltpu.async_copy(
                    table_hbm.at[idx_v.at[g * per_group + j]],
                    buf.at[pl.ds(j * CHUNK, CHUNK)],
                    gsem,
                )
                for j in range(per_group)
            ]
            for c in gathers:
                c.wait()
            out_copies[g] = pltpu.async_copy(
                buf.at[:, pl.ds(0, D)],
                out_hbm.at[pl.ds(base + g * GROUP, GROUP)],
                osem,
            )
        for g in range(max(0, NGROUP - 3), NGROUP):
            out_copies[g].wait()

    return gather_kernel


_sc_gather_cache = []


def _sc_gather(table, idx2):
    if not _sc_gather_cache:
        _sc_gather_cache.append(_make_sc_gather())
    return _sc_gather_cache[0](table, idx2)


def kernel(inpt, emb_mtrx):
    x = inpt.reshape(N, D)
    idx2, table, loss, perp = _distance_call(x, emb_mtrx)
    q = _sc_gather(table, idx2)
    q = q.reshape(inpt.shape)
    return (q, loss[0, 0], perp[0, 0])


# speedup vs baseline: 1.0706x; 1.0706x over previous
import functools

import jax
import jax.numpy as jnp
from jax import lax
from jax.experimental import pallas as pl
from jax.experimental.pallas import tpu as pltpu
from jax.experimental.pallas import tpu_sc as plsc

N = 32768
D = 64
K = 1024
TN = 4096
STEPS = N // TN
DP = 128

NW = 32
BPW = N // NW
CHUNK = 128
ROWS = BPW // CHUNK
GROUP = 128
NGROUP = BPW // GROUP


def _distance_body(x_ref, e_ref, idx_ref, tab_ref, loss_ref, perp_ref,
                   cnt_ref, acc_ref):
    step = pl.program_id(0)
    x = x_ref[...]
    e = e_ref[...]
    s = jnp.dot(-2.0 * x, e, preferred_element_type=jnp.float32)
    e2 = jnp.sum(e * e, axis=0, keepdims=True)
    half = s + e2
    m = jnp.min(half, axis=1, keepdims=True)
    ohm = half == m
    iota = lax.broadcasted_iota(jnp.int32, (TN, K), 1).astype(jnp.float32)
    idx = jnp.min(jnp.where(ohm, iota, float(K)), axis=1).astype(jnp.int32)
    oh = jnp.where(ohm, 1.0, 0.0)
    cnt_step = jnp.dot(jnp.full((1, TN), 1.0, jnp.float32), oh,
                       preferred_element_type=jnp.float32)
    part = jnp.sum(x * x) + jnp.sum(m)

    @pl.when(step == 0)
    def _init():
        cnt_ref[...] = jnp.zeros_like(cnt_ref)
        acc_ref[0] = 0.0
        et = jnp.swapaxes(e, 0, 1)
        tab_ref[...] = jnp.concatenate(
            [et, jnp.zeros((K, DP - D), jnp.float32)], axis=1)

    cnt_ref[...] += cnt_step
    acc_ref[0] += part
    idx_ref[...] = idx.reshape(TN // CHUNK, CHUNK)

    @pl.when(step == STEPS - 1)
    def _finish():
        loss_ref[...] = jnp.full((1, 1), acc_ref[0] / (N * D), jnp.float32)
        p = cnt_ref[...] * (1.0 / N)
        ent = jnp.sum(p * jnp.log(p + 1e-10))
        perp_ref[...] = jnp.full((1, 1), jnp.exp(-ent), jnp.float32)


def _distance_call(x, e):
    return pl.pallas_call(
        _distance_body,
        grid=(STEPS,),
        in_specs=[
            pl.BlockSpec((TN, D), lambda i: (i, 0)),
            pl.BlockSpec((D, K), lambda i: (0, 0)),
        ],
        out_specs=[
            pl.BlockSpec((TN // CHUNK, CHUNK), lambda i: (i, 0)),
            pl.BlockSpec((K, DP), lambda i: (0, 0)),
            pl.BlockSpec((1, 1), lambda i: (0, 0)),
            pl.BlockSpec((1, 1), lambda i: (0, 0)),
        ],
        out_shape=[
            jax.ShapeDtypeStruct((N // CHUNK, CHUNK), jnp.int32),
            jax.ShapeDtypeStruct((K, DP), jnp.float32),
            jax.ShapeDtypeStruct((1, 1), jnp.float32),
            jax.ShapeDtypeStruct((1, 1), jnp.float32),
        ],
        scratch_shapes=[
            pltpu.VMEM((1, K), jnp.float32),
            pltpu.SMEM((1,), jnp.float32),
        ],
    )(x, e)


def _make_sc_gather():
    mesh = plsc.VectorSubcoreMesh(core_axis_name="c", subcore_axis_name="s")

    @functools.partial(
        pl.kernel,
        mesh=mesh,
        out_type=jax.ShapeDtypeStruct((N, D), jnp.float32),
        scratch_types=[
            pltpu.VMEM((ROWS, CHUNK), jnp.int32),
            pltpu.VMEM((GROUP, DP), jnp.float32),
            pltpu.VMEM((GROUP, DP), jnp.float32),
            pltpu.VMEM((GROUP, D), jnp.float32),
            pltpu.VMEM((GROUP, D), jnp.float32),
            pltpu.SemaphoreType.DMA,
            pltpu.SemaphoreType.DMA,
        ],
    )
    def gather_kernel(table_hbm, idx_hbm, out_hbm, idx_v, bufa0, bufa1,
                      bufb0, bufb1, gsem, osem):
        wid = lax.axis_index("s") * 2 + lax.axis_index("c")
        base = wid * BPW
        pltpu.sync_copy(idx_hbm.at[pl.ds(wid * ROWS, ROWS)], idx_v)
        bufa = [bufa0, bufa1]
        bufb = [bufb0, bufb1]

        def start_gather(g):
            return pltpu.async_copy(
                table_hbm.at[idx_v.at[g]], bufa[g % 2], gsem)

        def compact(src, dst):
            def body(r0, _):
                for u in range(4):
                    r = r0 * 4 + u
                    for c in range(4):
                        dst[r, pl.ds(c * 16, 16)] = src[r, pl.ds(c * 16, 16)]
                return _
            lax.fori_loop(0, GROUP // 4, body, None, unroll=False)

        out_copies = [None] * NGROUP
        pending = start_gather(0)
        for g in range(NGROUP):
            pending.wait()
            if g + 1 < NGROUP:
                pending = start_gather(g + 1)
            if g >= 2:
                out_copies[g - 2].wait()
            compact(bufa[g % 2], bufb[g % 2])
            out_copies[g] = pltpu.async_copy(
                bufb[g % 2],
                out_hbm.at[pl.ds(base + g * GROUP, GROUP)],
                osem,
            )
        for g in range(NGROUP - 2, NGROUP):
            out_copies[g].wait()

    return gather_kernel


_sc_gather_cache = []


def _sc_gather(table, idx2):
    if not _sc_gather_cache:
        _sc_gather_cache.append(_make_sc_gather())
    return _sc_gather_cache[0](table, idx2)


def kernel(inpt, emb_mtrx):
    x = inpt.reshape(N, D)
    idx2, table, loss, perp = _distance_call(x, emb_mtrx)
    q = _sc_gather(table, idx2)
    q = q.reshape(inpt.shape)
    return (q, loss[0, 0], perp[0, 0])


# speedup vs baseline: 1.1789x; 1.1012x over previous
import functools

import jax
import jax.numpy as jnp
from jax import lax
from jax.experimental import pallas as pl
from jax.experimental.pallas import tpu as pltpu
from jax.experimental.pallas import tpu_sc as plsc

N = 32768
D = 64
K = 1024
TN = 4096
STEPS = N // TN
DP = 128

NW = 32
BPW = N // NW
CHUNK = 128
ROWS = BPW // CHUNK
GROUP = 128
NGROUP = BPW // GROUP


BB = 4
TB = 1024


def _distance_body(x_ref, e_ref, idx_ref, tab_ref, loss_ref, perp_ref,
                   cnt_ref, acc_ref):
    step = pl.program_id(0)
    e = e_ref[...]
    e2 = jnp.sum(e * e, axis=0, keepdims=True)

    @pl.when(step == 0)
    def _init():
        cnt_ref[...] = jnp.zeros_like(cnt_ref)
        acc_ref[0] = 0.0
        et = jnp.swapaxes(e, 0, 1)
        tab_ref[...] = jnp.concatenate(
            [et, jnp.zeros((K, DP - D), jnp.float32)], axis=1)

    iota = lax.broadcasted_iota(jnp.int32, (TB, K), 1).astype(jnp.float32)
    for b in range(BB):
        xb = x_ref[b]
        xbt = jnp.swapaxes(xb, 0, 1)
        s = jnp.dot(-2.0 * xbt, e,
                    preferred_element_type=jnp.float32)
        half = s + e2
        m = jnp.min(half, axis=1, keepdims=True)
        ohm = half == m
        idx = jnp.min(jnp.where(ohm, iota, float(K)), axis=1)
        oh = jnp.where(ohm, 1.0, 0.0)
        cnt_step = jnp.dot(jnp.full((1, TB), 1.0, jnp.float32), oh,
                           preferred_element_type=jnp.float32)
        part = jnp.sum(xb * xb) + jnp.sum(m)
        cnt_ref[...] += cnt_step
        acc_ref[0] += part
        idx_ref[pl.ds(b * (TB // CHUNK), TB // CHUNK), :] = (
            idx.astype(jnp.int32).reshape(TB // CHUNK, CHUNK))

    @pl.when(step == STEPS - 1)
    def _finish():
        loss_ref[...] = jnp.full((1, 1), acc_ref[0] / (N * D), jnp.float32)
        p = cnt_ref[...] * (1.0 / N)
        ent = jnp.sum(p * jnp.log(p + 1e-10))
        perp_ref[...] = jnp.full((1, 1), jnp.exp(-ent), jnp.float32)


def _distance_call(xt, e):
    steps = N // (BB * TB)
    return pl.pallas_call(
        _distance_body,
        grid=(steps,),
        in_specs=[
            pl.BlockSpec((BB, D, TB), lambda i: (i, 0, 0)),
            pl.BlockSpec((D, K), lambda i: (0, 0)),
        ],
        out_specs=[
            pl.BlockSpec((BB * TB // CHUNK, CHUNK), lambda i: (i, 0)),
            pl.BlockSpec((K, DP), lambda i: (0, 0)),
            pl.BlockSpec((1, 1), lambda i: (0, 0)),
            pl.BlockSpec((1, 1), lambda i: (0, 0)),
        ],
        out_shape=[
            jax.ShapeDtypeStruct((N // CHUNK, CHUNK), jnp.int32),
            jax.ShapeDtypeStruct((K, DP), jnp.float32),
            jax.ShapeDtypeStruct((1, 1), jnp.float32),
            jax.ShapeDtypeStruct((1, 1), jnp.float32),
        ],
        scratch_shapes=[
            pltpu.VMEM((1, K), jnp.float32),
            pltpu.SMEM((1,), jnp.float32),
        ],
    )(xt, e)


def _make_sc_gather():
    mesh = plsc.VectorSubcoreMesh(core_axis_name="c", subcore_axis_name="s")

    @functools.partial(
        pl.kernel,
        mesh=mesh,
        out_type=jax.ShapeDtypeStruct((N, D), jnp.float32),
        scratch_types=[
            pltpu.VMEM((ROWS, CHUNK), jnp.int32),
            pltpu.VMEM((GROUP, DP), jnp.float32),
            pltpu.VMEM((GROUP, DP), jnp.float32),
            pltpu.VMEM((GROUP, D), jnp.float32),
            pltpu.VMEM((GROUP, D), jnp.float32),
            pltpu.SemaphoreType.DMA,
            pltpu.SemaphoreType.DMA,
        ],
    )
    def gather_kernel(table_hbm, idx_hbm, out_hbm, idx_v, bufa0, bufa1,
                      bufb0, bufb1, gsem, osem):
        wid = lax.axis_index("s") * 2 + lax.axis_index("c")
        base = wid * BPW
        pltpu.sync_copy(idx_hbm.at[pl.ds(wid * ROWS, ROWS)], idx_v)
        bufa = [bufa0, bufa1]
        bufb = [bufb0, bufb1]

        def start_gather(g):
            return pltpu.async_copy(
                table_hbm.at[idx_v.at[g]], bufa[g % 2], gsem)

        def compact(src, dst):
            def body(r0, _):
                for u in range(4):
                    r = r0 * 4 + u
                    for c in range(4):
                        dst[r, pl.ds(c * 16, 16)] = src[r, pl.ds(c * 16, 16)]
                return _
            lax.fori_loop(0, GROUP // 4, body, None, unroll=False)

        out_copies = [None] * NGROUP
        pending = start_gather(0)
        for g in range(NGROUP):
            pending.wait()
            if g + 1 < NGROUP:
                pending = start_gather(g + 1)
            if g >= 2:
                out_copies[g - 2].wait()
            compact(bufa[g % 2], bufb[g % 2])
            out_copies[g] = pltpu.async_copy(
                bufb[g % 2],
                out_hbm.at[pl.ds(base + g * GROUP, GROUP)],
                osem,
            )
        for g in range(NGROUP - 2, NGROUP):
            out_copies[g].wait()

    return gather_kernel


_sc_gather_cache = []


def _sc_gather(table, idx2):
    if not _sc_gather_cache:
        _sc_gather_cache.append(_make_sc_gather())
    return _sc_gather_cache[0](table, idx2)


def kernel(inpt, emb_mtrx):
    xt = jnp.swapaxes(inpt, 1, 2)
    idx2, table, loss, perp = _distance_call(xt, emb_mtrx)
    q = _sc_gather(table, idx2)
    q = q.reshape(inpt.shape)
    return (q, loss[0, 0], perp[0, 0])
